# Initial kernel scaffold; baseline (speedup 1.0000x reference)
import jax
import jax.numpy as jnp
from jax.experimental import pallas as pl


def kernel(tokens, table):
    raise NotImplementedError("write your pallas kernel here")


# speedup vs baseline: 2.4200x; 2.4200x over previous
import functools
import math

import jax
import jax.numpy as jnp
from jax import lax
from jax.experimental import pallas as pl
from jax.experimental.pallas import tpu as pltpu
from jax.experimental.pallas import tpu_sc as plsc

VOCAB = 100000
EMB = 128
SCALE = math.sqrt(float(EMB))

NC = 2
NS = 16
NW = NC * NS

CHUNK = 128
B = 4096 * 50
NCHUNK = B // (NW * CHUNK)


def _sc_body(idx_hbm, table_hbm, out_hbm, idx_v, rows_v, sem):
    wid = lax.axis_index("s") * NC + lax.axis_index("c")
    pltpu.sync_copy(idx_hbm.at[wid], idx_v)

    def chunk_body(c, carry):
        pltpu.async_copy(table_hbm.at[idx_v.at[c]], rows_v, sem).wait()

        def row_body(r, carry2):
            for j in range(EMB // 16):
                sl = pl.ds(j * 16, 16)
                rows_v[r, sl] = rows_v[r, sl] * SCALE
            return carry2

        lax.fori_loop(0, CHUNK, row_body, 0)
        out_base = pl.multiple_of((wid * NCHUNK + c) * CHUNK, CHUNK)
        pltpu.sync_copy(rows_v, out_hbm.at[pl.ds(out_base, CHUNK)])
        return carry

    lax.fori_loop(0, NCHUNK, chunk_body, 0)


@functools.partial(jax.jit, static_argnames=())
def _sc_embed(idx2d, table):
    mesh = plsc.VectorSubcoreMesh(core_axis_name="c", subcore_axis_name="s")
    run = pl.kernel(
        _sc_body,
        out_type=jax.ShapeDtypeStruct((B, EMB), jnp.float32),
        mesh=mesh,
        scratch_types=[
            pltpu.VMEM((NCHUNK, CHUNK), jnp.int32),
            pltpu.VMEM((CHUNK, EMB), jnp.float32),
            pltpu.SemaphoreType.DMA,
        ],
    )
    return run(idx2d, table)


def kernel(tokens, table):
    idx2d = tokens.reshape(NW, NCHUNK, CHUNK)
    out = _sc_embed(idx2d, table)
    return out.reshape(tokens.shape[0], tokens.shape[1], EMB)


# speedup vs baseline: 2.9395x; 1.2146x over previous
import functools
import math

import jax
import jax.numpy as jnp
from jax import lax
from jax.experimental import pallas as pl
from jax.experimental.pallas import tpu as pltpu
from jax.experimental.pallas import tpu_sc as plsc

VOCAB = 100000
EMB = 128
SCALE = math.sqrt(float(EMB))

NC = 2
NS = 16
NW = NC * NS

CHUNK = 128
B = 4096 * 50
NCHUNK = B // (NW * CHUNK)
NBUF = 5
NGRP = NCHUNK // NBUF
ROWS_PER_ITER = 4


def _sc_body(idx_hbm, table_hbm, out_hbm, idx_v, bufs, *sems):
    gsem = sems[:NBUF]
    ssem = sems[NBUF:]
    wid = lax.axis_index("s") * NC + lax.axis_index("c")
    pltpu.sync_copy(idx_hbm.at[wid], idx_v)
    chunk0 = wid * NCHUNK

    def gather_start(c, b):
        pltpu.async_copy(table_hbm.at[idx_v.at[c]], bufs.at[b], gsem[b])

    def gather_wait(c, b):
        pltpu.make_async_copy(table_hbm.at[idx_v.at[c]], bufs.at[b], gsem[b]).wait()

    def out_slot(c):
        return out_hbm.at[pl.ds(pl.multiple_of((chunk0 + c) * CHUNK, CHUNK), CHUNK)]

    def scatter_start(c, b):
        pltpu.async_copy(bufs.at[b], out_slot(c), ssem[b])

    def scatter_wait(c, b):
        pltpu.make_async_copy(bufs.at[b], out_slot(c), ssem[b]).wait()

    def scale_buf(b):
        def row_body(r, carry):
            for rr in range(ROWS_PER_ITER):
                for j in range(EMB // 16):
                    sl = pl.ds(j * 16, 16)
                    bufs[b, r * ROWS_PER_ITER + rr, sl] = (
                        bufs[b, r * ROWS_PER_ITER + rr, sl] * SCALE)
            return carry

        lax.fori_loop(0, CHUNK // ROWS_PER_ITER, row_body, 0)

    for b in range(NBUF):
        gather_start(b, b)

    def group_body(g, carry):
        cg = g * NBUF
        for b in range(NBUF):
            gather_wait(cg + b, b)
            scale_buf(b)
            scatter_start(cg + b, b)
        for b in range(NBUF):
            scatter_wait(cg + b, b)
            gather_start(cg + NBUF + b, b)
        return carry

    lax.fori_loop(0, NGRP - 1, group_body, 0)

    cg = (NGRP - 1) * NBUF
    for b in range(NBUF):
        gather_wait(cg + b, b)
        scale_buf(b)
        scatter_start(cg + b, b)
    for b in range(NBUF):
        scatter_wait(cg + b, b)


@jax.jit
def _sc_embed(idx3d, table):
    mesh = plsc.VectorSubcoreMesh(core_axis_name="c", subcore_axis_name="s")
    run = pl.kernel(
        _sc_body,
        out_type=jax.ShapeDtypeStruct((B, EMB), jnp.float32),
        mesh=mesh,
        scratch_types=[
            pltpu.VMEM((NCHUNK, CHUNK), jnp.int32),
            pltpu.VMEM((NBUF, CHUNK, EMB), jnp.float32),
        ] + [pltpu.SemaphoreType.DMA] * (2 * NBUF),
    )
    return run(idx3d, table)


def kernel(tokens, table):
    idx3d = tokens.reshape(NW, NCHUNK, CHUNK)
    out = _sc_embed(idx3d, table)
    return out.reshape(tokens.shape[0], tokens.shape[1], EMB)
